# Initial kernel scaffold; baseline (speedup 1.0000x reference)
import jax
import jax.numpy as jnp
from jax.experimental import pallas as pl


def kernel(x, edge_index, batch, W1, b1, g1, be1, W2, b2, g2, be2, W3, b3, g3, be3, W4, b4, g4, be4, fcW1, fcb1, fcW2, fcb2):
    raise NotImplementedError("write your pallas kernel here")


# speedup vs baseline: 12.2239x; 12.2239x over previous
import functools

import jax
import jax.numpy as jnp
from jax import lax
from jax.experimental import pallas as pl
from jax.experimental.pallas import tpu as pltpu
from jax.experimental.pallas import tpu_sc as plsc

N = 100000
NP = 100352
E = 1600000
EPAD = 1605632
EB = EPAD // 128
NG = 64
D = 64
NC, NS, L = 2, 16, 16
RS = NP // NS
KB = 8
SPMM_OUT = EB // NS // KB
HALF_OUT = EB // (NC * NS) // KB
ZR = 392
RB = 2048
GRID = NP // RB
EPS = 1e-5

_f32 = jnp.float32


@functools.lru_cache(maxsize=None)
def _sc_mesh():
    return plsc.VectorSubcoreMesh(core_axis_name="c", subcore_axis_name="s",
                                  num_cores=NC, num_subcores=NS)


_sc_params = pltpu.CompilerParams(use_tc_tiling_on_sc=False)


def _deg_body(dst2d, out, idxd, ones_v, zero_v, acc):
    cid = lax.axis_index("c")
    sid = lax.axis_index("s")

    def fz(i, _):
        zero_v[pl.ds(i * L, L)] = jnp.zeros((L,), _f32)
        return 0
    lax.fori_loop(0, 784 // L, fz, 0)

    def fo(i, _):
        ones_v[pl.ds(i * L, L)] = jnp.ones((L,), _f32)
        return 0
    lax.fori_loop(0, 128 // L, fo, 0)

    def zc(i, _):
        pltpu.sync_copy(zero_v, acc.at[pl.ds(sid * RS + i * 784, 784)])
        return 0
    lax.fori_loop(0, RS // 784, zc, 0)
    plsc.subcore_barrier()

    base = (cid * NS + sid) * (HALF_OUT * KB)

    def outer(g, _):
        pltpu.sync_copy(dst2d.at[pl.ds(base + g * KB, KB)], idxd)
        for t in range(KB):
            pltpu.sync_copy(ones_v, acc.at[idxd.at[t]], add=True)
        return 0
    lax.fori_loop(0, HALF_OUT, outer, 0)
    plsc.subcore_barrier()
    pltpu.sync_copy(acc.at[pl.ds(sid * RS, RS)],
                    out.at[cid].at[pl.ds(sid * RS, RS)])


@functools.lru_cache(maxsize=None)
def _deg_kernel():
    return pl.kernel(
        _deg_body,
        out_type=jax.ShapeDtypeStruct((NC, NP), _f32),
        mesh=_sc_mesh(),
        scratch_types=[
            pltpu.VMEM((KB, 128), jnp.int32),
            pltpu.VMEM((128,), _f32),
            pltpu.VMEM((784,), _f32),
            pltpu.VMEM_SHARED((NP,), _f32),
        ],
    )


def _deg_call(dstp):
    return _deg_kernel()(dstp)


def _spmv_body(xdtab, src2d, dst2d, out, idxs, idxd, rows, zero_v, acc, sem):
    cid = lax.axis_index("c")
    sid = lax.axis_index("s")

    def fz(i, _):
        zero_v[pl.ds(i * L, L)] = jnp.zeros((L,), _f32)
        return 0
    lax.fori_loop(0, 784 // L, fz, 0)

    def zc(i, _):
        pltpu.sync_copy(zero_v, acc.at[pl.ds(sid * RS + i * 784, 784)])
        return 0
    lax.fori_loop(0, RS // 784, zc, 0)
    plsc.subcore_barrier()

    base = (cid * NS + sid) * (HALF_OUT * KB)

    def outer(g, _):
        pltpu.sync_copy(src2d.at[pl.ds(base + g * KB, KB)], idxs)
        pltpu.sync_copy(dst2d.at[pl.ds(base + g * KB, KB)], idxd)
        descs = [pltpu.async_copy(xdtab.at[idxs.at[t]], rows.at[t], sem)
                 for t in range(KB)]
        for d in descs:
            d.wait()
        for t in range(KB):
            pltpu.sync_copy(rows.at[t], acc.at[idxd.at[t]], add=True)
        return 0
    lax.fori_loop(0, HALF_OUT, outer, 0)
    plsc.subcore_barrier()
    pltpu.sync_copy(acc.at[pl.ds(sid * RS, RS)],
                    out.at[cid].at[pl.ds(sid * RS, RS)])


@functools.lru_cache(maxsize=None)
def _spmv_kernel():
    return pl.kernel(
        _spmv_body,
        out_type=jax.ShapeDtypeStruct((NC, NP), _f32),
        mesh=_sc_mesh(),
        scratch_types=[
            pltpu.VMEM((KB, 128), jnp.int32),
            pltpu.VMEM((KB, 128), jnp.int32),
            pltpu.VMEM((KB, 128), _f32),
            pltpu.VMEM((784,), _f32),
            pltpu.VMEM_SHARED((NP,), _f32),
            pltpu.SemaphoreType.DMA,
        ],
    )


def _spmv_call(xdtab, srcp, dstp):
    return _spmv_kernel()(xdtab, srcp, dstp)


def _spmm_body(gtab, src2d, dst2d, out, idxs, idxd, rows, zero_v, acc, sem):
    cid = lax.axis_index("c")
    sid = lax.axis_index("s")

    def fz(i, _):
        zero_v[i, :] = jnp.zeros((L,), _f32)
        return 0
    lax.fori_loop(0, ZR, fz, 0)

    for kk in range(2):
        chunk = 2 * cid + kk
        gt = gtab.at[chunk]
        ot = out.at[chunk]

        def zc(i, _):
            pltpu.sync_copy(zero_v, acc.at[pl.ds(sid * RS + i * ZR, ZR)])
            return 0
        lax.fori_loop(0, RS // ZR, zc, 0)
        plsc.subcore_barrier()

        base = sid * (SPMM_OUT * KB)

        def outer(g, _):
            pltpu.sync_copy(src2d.at[pl.ds(base + g * KB, KB)], idxs)
            pltpu.sync_copy(dst2d.at[pl.ds(base + g * KB, KB)], idxd)
            descs = [pltpu.async_copy(gt.at[idxs.at[t]], rows.at[t], sem)
                     for t in range(KB)]
            for d in descs:
                d.wait()
            for t in range(KB):
                pltpu.sync_copy(rows.at[t], acc.at[idxd.at[t]], add=True)
            return 0
        lax.fori_loop(0, SPMM_OUT, outer, 0)
        plsc.subcore_barrier()
        pltpu.sync_copy(acc.at[pl.ds(sid * RS, RS)],
                        ot.at[pl.ds(sid * RS, RS)])


@functools.lru_cache(maxsize=None)
def _spmm_kernel():
    return pl.kernel(
        _spmm_body,
        out_type=jax.ShapeDtypeStruct((4, NP, L), _f32),
        mesh=_sc_mesh(),
        scratch_types=[
            pltpu.VMEM((KB, 128), jnp.int32),
            pltpu.VMEM((KB, 128), jnp.int32),
            pltpu.VMEM((KB, 128, L), _f32),
            pltpu.VMEM((ZR, L), _f32),
            pltpu.VMEM_SHARED((NP, L), _f32),
            pltpu.SemaphoreType.DMA,
        ],
        compiler_params=_sc_params,
    )


def _spmm_call(gtab, srcp, dstp):
    return _spmm_kernel()(gtab, srcp, dstp)


def _tc1_body(degp_ref, x_ref, dinv_ref, xd_ref):
    deg = degp_ref[0] + degp_ref[1] + 1.0
    dinv = lax.rsqrt(deg)
    dinv_ref[...] = dinv
    xd_ref[...] = x_ref[...] * dinv


def _tc1(degp, x_p):
    return pl.pallas_call(
        _tc1_body,
        grid=(GRID,),
        in_specs=[
            pl.BlockSpec((2, RB, 1), lambda i: (0, i, 0)),
            pl.BlockSpec((RB, 1), lambda i: (i, 0)),
        ],
        out_specs=[
            pl.BlockSpec((RB, 1), lambda i: (i, 0)),
            pl.BlockSpec((RB, 1), lambda i: (i, 0)),
        ],
        out_shape=[
            jax.ShapeDtypeStruct((NP, 1), _f32),
            jax.ShapeDtypeStruct((NP, 1), _f32),
        ],
    )(degp, x_p)


def _tc2_body(s_ref, xd_ref, dinv_ref, a_ref, sa_ref, sa2_ref):
    i = pl.program_id(0)
    a = dinv_ref[...] * (s_ref[0] + s_ref[1] + xd_ref[...])
    a_ref[...] = a
    rows = i * RB + lax.broadcasted_iota(jnp.int32, (RB, 1), 0)
    am = jnp.where(rows < N, a, 0.0)

    @pl.when(i == 0)
    def _():
        sa_ref[0, 0] = 0.0
        sa2_ref[0, 0] = 0.0

    sa_ref[0, 0] += jnp.sum(am)
    sa2_ref[0, 0] += jnp.sum(am * am)


def _tc2(sp, xd, dinv):
    return pl.pallas_call(
        _tc2_body,
        grid=(GRID,),
        in_specs=[
            pl.BlockSpec((2, RB, 1), lambda i: (0, i, 0)),
            pl.BlockSpec((RB, 1), lambda i: (i, 0)),
            pl.BlockSpec((RB, 1), lambda i: (i, 0)),
        ],
        out_specs=[
            pl.BlockSpec((RB, 1), lambda i: (i, 0)),
            pl.BlockSpec((1, 1), lambda i: (0, 0), memory_space=pltpu.SMEM),
            pl.BlockSpec((1, 1), lambda i: (0, 0), memory_space=pltpu.SMEM),
        ],
        out_shape=[
            jax.ShapeDtypeStruct((NP, 1), _f32),
            jax.ShapeDtypeStruct((1, 1), _f32),
            jax.ShapeDtypeStruct((1, 1), _f32),
        ],
    )(sp, xd, dinv)


def _tc3_body(a_ref, sa_ref, sa2_ref, w1_ref, g1_ref, be1_ref, w2_ref,
              dinv_ref, g_ref):
    m = sa_ref[0, 0] / N
    v = sa2_ref[0, 0] / N - m * m
    w1 = w1_ref[...]
    alpha = w1 * g1_ref[...] * lax.rsqrt(v * w1 * w1 + EPS)
    shift = be1_ref[...] - m * alpha
    h1 = jax.nn.relu(a_ref[...] * alpha + shift)
    z = jnp.dot(h1, w2_ref[...], preferred_element_type=_f32)
    g = dinv_ref[...] * z
    for c in range(4):
        g_ref[c] = g[:, c * L:(c + 1) * L]


def _tc3(a, sa, sa2, w1row, g1, be1, W2, dinv):
    return pl.pallas_call(
        _tc3_body,
        grid=(GRID,),
        in_specs=[
            pl.BlockSpec((RB, 1), lambda i: (i, 0)),
            pl.BlockSpec((1, 1), lambda i: (0, 0), memory_space=pltpu.SMEM),
            pl.BlockSpec((1, 1), lambda i: (0, 0), memory_space=pltpu.SMEM),
            pl.BlockSpec((1, D), lambda i: (0, 0)),
            pl.BlockSpec((1, D), lambda i: (0, 0)),
            pl.BlockSpec((1, D), lambda i: (0, 0)),
            pl.BlockSpec((D, D), lambda i: (0, 0)),
            pl.BlockSpec((RB, 1), lambda i: (i, 0)),
        ],
        out_specs=pl.BlockSpec((4, RB, L), lambda i: (0, i, 0)),
        out_shape=jax.ShapeDtypeStruct((4, NP, L), _f32),
    )(a, sa, sa2, w1row, g1, be1, W2, dinv)


def _tc4_body(aggc_ref, gc_ref, dinv_ref, b_ref, pre_ref, s_ref, s2_ref):
    i = pl.program_id(0)
    agg = jnp.concatenate([aggc_ref[c] for c in range(4)], axis=-1)
    g = jnp.concatenate([gc_ref[c] for c in range(4)], axis=-1)
    pre = dinv_ref[...] * (agg + g) + b_ref[...]
    pre_ref[...] = pre
    rows = i * RB + lax.broadcasted_iota(jnp.int32, (RB, 1), 0)
    pm = jnp.where(rows < N, pre, 0.0)

    @pl.when(i == 0)
    def _():
        s_ref[...] = jnp.zeros_like(s_ref)
        s2_ref[...] = jnp.zeros_like(s2_ref)

    s_ref[...] += jnp.sum(pm, axis=0, keepdims=True)
    s2_ref[...] += jnp.sum(pm * pm, axis=0, keepdims=True)


def _tc4(aggc, gc, dinv, b):
    return pl.pallas_call(
        _tc4_body,
        grid=(GRID,),
        in_specs=[
            pl.BlockSpec((4, RB, L), lambda i: (0, i, 0)),
            pl.BlockSpec((4, RB, L), lambda i: (0, i, 0)),
            pl.BlockSpec((RB, 1), lambda i: (i, 0)),
            pl.BlockSpec((1, D), lambda i: (0, 0)),
        ],
        out_specs=[
            pl.BlockSpec((RB, D), lambda i: (i, 0)),
            pl.BlockSpec((1, D), lambda i: (0, 0)),
            pl.BlockSpec((1, D), lambda i: (0, 0)),
        ],
        out_shape=[
            jax.ShapeDtypeStruct((NP, D), _f32),
            jax.ShapeDtypeStruct((1, D), _f32),
            jax.ShapeDtypeStruct((1, D), _f32),
        ],
    )(aggc, gc, dinv, b)


def _bn_scale_shift(s_ref, s2_ref, ga_ref, be_ref):
    m = s_ref[...] / N
    v = s2_ref[...] / N - m * m
    scale = ga_ref[...] * lax.rsqrt(v + EPS)
    shift = be_ref[...] - m * scale
    return scale, shift


def _tc5_body(pre_ref, s_ref, s2_ref, ga_ref, be_ref, w_ref, dinv_ref, g_ref):
    scale, shift = _bn_scale_shift(s_ref, s2_ref, ga_ref, be_ref)
    h = jax.nn.relu(pre_ref[...] * scale + shift)
    z = jnp.dot(h, w_ref[...], preferred_element_type=_f32)
    g = dinv_ref[...] * z
    for c in range(4):
        g_ref[c] = g[:, c * L:(c + 1) * L]


def _tc5(pre, s, s2, ga, be, W, dinv):
    return pl.pallas_call(
        _tc5_body,
        grid=(GRID,),
        in_specs=[
            pl.BlockSpec((RB, D), lambda i: (i, 0)),
            pl.BlockSpec((1, D), lambda i: (0, 0)),
            pl.BlockSpec((1, D), lambda i: (0, 0)),
            pl.BlockSpec((1, D), lambda i: (0, 0)),
            pl.BlockSpec((1, D), lambda i: (0, 0)),
            pl.BlockSpec((D, D), lambda i: (0, 0)),
            pl.BlockSpec((RB, 1), lambda i: (i, 0)),
        ],
        out_specs=pl.BlockSpec((4, RB, L), lambda i: (0, i, 0)),
        out_shape=jax.ShapeDtypeStruct((4, NP, L), _f32),
    )(pre, s, s2, ga, be, W, dinv)


def _tc6_body(pre_ref, s_ref, s2_ref, ga_ref, be_ref, batch_ref,
              psum_ref, cnt_ref):
    i = pl.program_id(0)
    scale, shift = _bn_scale_shift(s_ref, s2_ref, ga_ref, be_ref)
    h = jax.nn.relu(pre_ref[...] * scale + shift)
    onehot = (batch_ref[...] ==
              lax.broadcasted_iota(jnp.int32, (RB, NG), 1)).astype(_f32)

    @pl.when(i == 0)
    def _():
        psum_ref[...] = jnp.zeros_like(psum_ref)
        cnt_ref[...] = jnp.zeros_like(cnt_ref)

    psum_ref[...] += lax.dot_general(
        onehot, h, (((0,), (0,)), ((), ())), preferred_element_type=_f32)
    cnt_ref[...] += lax.dot_general(
        onehot, jnp.ones((RB, 1), _f32), (((0,), (0,)), ((), ())),
        preferred_element_type=_f32)


def _tc6(pre, s, s2, ga, be, batch_p):
    return pl.pallas_call(
        _tc6_body,
        grid=(GRID,),
        in_specs=[
            pl.BlockSpec((RB, D), lambda i: (i, 0)),
            pl.BlockSpec((1, D), lambda i: (0, 0)),
            pl.BlockSpec((1, D), lambda i: (0, 0)),
            pl.BlockSpec((1, D), lambda i: (0, 0)),
            pl.BlockSpec((1, D), lambda i: (0, 0)),
            pl.BlockSpec((RB, 1), lambda i: (i, 0)),
        ],
        out_specs=[
            pl.BlockSpec((NG, D), lambda i: (0, 0)),
            pl.BlockSpec((NG, 1), lambda i: (0, 0)),
        ],
        out_shape=[
            jax.ShapeDtypeStruct((NG, D), _f32),
            jax.ShapeDtypeStruct((NG, 1), _f32),
        ],
    )(pre, s, s2, ga, be, batch_p)


def _tc7_body(psum_ref, cnt_ref, w1_ref, b1_ref, w2_ref, b2_ref, out_ref):
    pooled = psum_ref[...] / jnp.maximum(cnt_ref[...], 1.0)
    h = jax.nn.relu(jnp.dot(pooled, w1_ref[...], preferred_element_type=_f32)
                    + b1_ref[...])
    out_ref[...] = jnp.dot(h, w2_ref[...], preferred_element_type=_f32) \
        + b2_ref[...]


def _tc7(psum, cnt, fcW1, fcb1, fcW2, fcb2):
    return pl.pallas_call(
        _tc7_body,
        out_shape=jax.ShapeDtypeStruct((NG, 10), _f32),
    )(psum, cnt, fcW1, fcb1, fcW2, fcb2)


def kernel(x, edge_index, batch, W1, b1, g1, be1, W2, b2, g2, be2,
           W3, b3, g3, be3, W4, b4, g4, be4, fcW1, fcb1, fcW2, fcb2):
    i32 = jnp.int32
    src = edge_index[0].astype(i32)
    dst = edge_index[1].astype(i32)
    padE = EPAD - E
    srcp = jnp.concatenate([src, jnp.full((padE,), N, i32)]).reshape(EB, 128)
    dstp = jnp.concatenate([dst, jnp.full((padE,), N, i32)]).reshape(EB, 128)

    degp = _deg_call(dstp).reshape(NC, NP, 1)
    x_p = jnp.pad(x[:, 0], (0, NP - N)).reshape(NP, 1)
    dinv, xd = _tc1(degp, x_p)

    sp = _spmv_call(xd.reshape(NP), srcp, dstp).reshape(NC, NP, 1)
    a, sa, sa2 = _tc2(sp, xd, dinv)

    g2c = _tc3(a, sa, sa2, W1[0:1], g1.reshape(1, D), be1.reshape(1, D),
               W2, dinv)
    agg2 = _spmm_call(g2c, srcp, dstp)
    pre2, s2s, s2q = _tc4(agg2, g2c, dinv, b2.reshape(1, D))

    g3c = _tc5(pre2, s2s, s2q, g2.reshape(1, D), be2.reshape(1, D), W3, dinv)
    agg3 = _spmm_call(g3c, srcp, dstp)
    pre3, s3s, s3q = _tc4(agg3, g3c, dinv, b3.reshape(1, D))

    g4c = _tc5(pre3, s3s, s3q, g3.reshape(1, D), be3.reshape(1, D), W4, dinv)
    agg4 = _spmm_call(g4c, srcp, dstp)
    pre4, s4s, s4q = _tc4(agg4, g4c, dinv, b4.reshape(1, D))

    batch_p = jnp.pad(batch.astype(i32), (0, NP - N),
                      constant_values=NG).reshape(NP, 1)
    psum, cnt = _tc6(pre4, s4s, s4q, g4.reshape(1, D), be4.reshape(1, D),
                     batch_p)

    return _tc7(psum, cnt, fcW1, fcb1.reshape(1, 128), fcW2,
                fcb2.reshape(1, 10))
